# Initial kernel scaffold; baseline (speedup 1.0000x reference)
import functools

import jax
import jax.numpy as jnp
from jax import lax
from jax.experimental import pallas as pl
from jax.experimental.pallas import tpu as pltpu
from jax.experimental.pallas import tpu_sc as plsc

_N = 16384
_HIDDEN = 768
_DIM = 256
_N_EMBED = 8192

_BN = 256
_NUM_BLOCKS = _N // _BN


def _tc_body(x_ref, wt_ref, b_ref, embed_ref, ind_ref, diff_ref):
    i = pl.program_id(0)

    xe = jnp.dot(x_ref[...], wt_ref[...], preferred_element_type=jnp.float32)
    xe = xe + b_ref[...]

    emb = embed_ref[...]
    s = jnp.dot(xe, emb, preferred_element_type=jnp.float32)
    e2 = jnp.sum(emb * emb, axis=0, keepdims=True)
    xe2 = jnp.sum(xe * xe, axis=1, keepdims=True)
    dist = xe2 - 2.0 * s + e2

    m = jnp.min(dist, axis=1, keepdims=True)
    iota = lax.broadcasted_iota(jnp.int32, dist.shape, 1)
    ind = jnp.min(jnp.where(dist <= m, iota, _N_EMBED), axis=1)
    ind_ref[...] = ind.reshape(1, 1, _BN)

    @pl.when(i == 0)
    def _init():
        diff_ref[...] = jnp.zeros_like(diff_ref)

    diff_ref[...] += jnp.sum(m).reshape(1, 1) * (1.0 / (_N * _DIM))


def _tc_search(x, wt, b2, embed):
    return pl.pallas_call(
        _tc_body,
        grid=(_NUM_BLOCKS,),
        in_specs=[
            pl.BlockSpec((_BN, _HIDDEN), lambda i: (i, 0)),
            pl.BlockSpec((_HIDDEN, _DIM), lambda i: (0, 0)),
            pl.BlockSpec((1, _DIM), lambda i: (0, 0)),
            pl.BlockSpec((_DIM, _N_EMBED), lambda i: (0, 0)),
        ],
        out_specs=[
            pl.BlockSpec((1, 1, _BN), lambda i: (i, 0, 0)),
            pl.BlockSpec((1, 1), lambda i: (0, 0)),
        ],
        out_shape=[
            jax.ShapeDtypeStruct((_NUM_BLOCKS, 1, _BN), jnp.int32),
            jax.ShapeDtypeStruct((1, 1), jnp.float32),
        ],
        compiler_params=pltpu.CompilerParams(
            dimension_semantics=("arbitrary",),
        ),
    )(x, wt, b2, embed)


_SC_INFO = plsc.get_sparse_core_info()
_NC = _SC_INFO.num_cores
_NS = _SC_INFO.num_subcores
_NW = _NC * _NS
_B_PER_W = _N // _NW
_CHUNK = 128
_NCHUNK = _B_PER_W // _CHUNK


def _sc_gather(table, idx):
    mesh = plsc.VectorSubcoreMesh(core_axis_name="c", subcore_axis_name="s")

    @functools.partial(
        pl.kernel,
        mesh=mesh,
        out_type=jax.ShapeDtypeStruct((_N, _DIM), jnp.float32),
        scratch_types=[
            pltpu.VMEM((_CHUNK,), jnp.int32),
            pltpu.VMEM((_CHUNK, _DIM), jnp.float32),
            pltpu.SemaphoreType.DMA,
        ],
    )
    def gather_k(table_hbm, idx_hbm, out_hbm, idx_v, rows_v, sem):
        wid = lax.axis_index("s") * _NC + lax.axis_index("c")
        base = wid * _B_PER_W
        for c in range(_NCHUNK):
            lo = base + c * _CHUNK
            pltpu.sync_copy(idx_hbm.at[pl.ds(lo, _CHUNK)], idx_v)
            pltpu.async_copy(table_hbm.at[idx_v], rows_v, sem).wait()
            pltpu.sync_copy(rows_v, out_hbm.at[pl.ds(lo, _CHUNK)])

    return gather_k(table, idx)


def kernel(x, W, b, embed):
    wt = W.T
    b2 = b.reshape(1, _DIM)
    ind_blocks, diff = _tc_search(x, wt, b2, embed)
    embed_ind = ind_blocks.reshape(_N)
    quantize = _sc_gather(embed.T, embed_ind)
    return (quantize, diff[0, 0], embed_ind)


# speedup vs baseline: 1.2051x; 1.2051x over previous
import functools

import jax
import jax.numpy as jnp
from jax import lax
from jax.experimental import pallas as pl
from jax.experimental.pallas import tpu as pltpu
from jax.experimental.pallas import tpu_sc as plsc

_N = 16384
_HIDDEN = 768
_DIM = 256
_N_EMBED = 8192

_BN = 256
_NUM_BLOCKS = _N // _BN
_CHUNKS = ((0, 2816), (2816, 5632), (5632, 8192))


def _tc_body(x_ref, wt_ref, b_ref, embed_ref, ind_ref, diff_ref):
    i = pl.program_id(0)
    bf = jnp.bfloat16

    def dotc(lo, sz):
        return jnp.dot(x_ref[:, lo:lo + sz].astype(bf),
                       wt_ref[lo:lo + sz, :].astype(bf),
                       preferred_element_type=jnp.float32)

    xe = dotc(0, 256) + dotc(256, 256) + dotc(512, 256)
    xe = xe + b_ref[...]

    xe_b = xe.astype(bf)
    xe2 = jnp.sum(xe * xe, axis=1, keepdims=True)

    acc_v = acc_i = acc_raw = None
    for lo, hi in _CHUNKS:
        emb = embed_ref[:, lo:hi]
        s = jnp.dot(xe_b, emb.astype(bf), preferred_element_type=jnp.float32)
        e2 = jnp.sum(emb * emb, axis=0, keepdims=True)
        dist = (xe2 - 2.0 * s) + e2
        m = jnp.min(dist, axis=1, keepdims=True)
        iota = lax.broadcasted_iota(jnp.int32, dist.shape, 1) + lo
        idx = jnp.min(jnp.where(dist <= m, iota, _N_EMBED), axis=1,
                      keepdims=True)
        mq = m.astype(bf).astype(jnp.float32)
        if acc_v is None:
            acc_v, acc_i, acc_raw = mq, idx, m
        else:
            keep = acc_v <= m
            acc_i = jnp.where(keep, acc_i, idx)
            acc_raw = jnp.where(keep, acc_raw, m)
            acc_v = jnp.where(keep, acc_v, mq)

    ind_ref[...] = acc_i.reshape(1, 1, _BN)

    @pl.when(i == 0)
    def _init():
        diff_ref[...] = jnp.zeros_like(diff_ref)

    diff_ref[...] += jnp.sum(acc_raw).reshape(1, 1) * (1.0 / (_N * _DIM))


def _tc_search(x, wt, b2, embed):
    return pl.pallas_call(
        _tc_body,
        grid=(_NUM_BLOCKS,),
        in_specs=[
            pl.BlockSpec((_BN, _HIDDEN), lambda i: (i, 0)),
            pl.BlockSpec((_HIDDEN, _DIM), lambda i: (0, 0)),
            pl.BlockSpec((1, _DIM), lambda i: (0, 0)),
            pl.BlockSpec((_DIM, _N_EMBED), lambda i: (0, 0)),
        ],
        out_specs=[
            pl.BlockSpec((1, 1, _BN), lambda i: (i, 0, 0)),
            pl.BlockSpec((1, 1), lambda i: (0, 0)),
        ],
        out_shape=[
            jax.ShapeDtypeStruct((_NUM_BLOCKS, 1, _BN), jnp.int32),
            jax.ShapeDtypeStruct((1, 1), jnp.float32),
        ],
        compiler_params=pltpu.CompilerParams(
            dimension_semantics=("arbitrary",),
        ),
    )(x, wt, b2, embed)


try:
    _SC_INFO = plsc.get_sparse_core_info()
    _NC, _NS = _SC_INFO.num_cores, _SC_INFO.num_subcores
except Exception:
    _NC, _NS = 2, 16
_NW = _NC * _NS
_B_PER_W = _N // _NW
_CHUNK = 128
_NCHUNK = _B_PER_W // _CHUNK


def _sc_gather(table, idx):
    mesh = plsc.VectorSubcoreMesh(core_axis_name="c", subcore_axis_name="s")

    @functools.partial(
        pl.kernel,
        mesh=mesh,
        out_type=jax.ShapeDtypeStruct((_N, _DIM), jnp.float32),
        scratch_types=[
            pltpu.VMEM((_CHUNK,), jnp.int32),
            pltpu.VMEM((_CHUNK, _DIM), jnp.float32),
            pltpu.SemaphoreType.DMA,
        ],
    )
    def gather_k(table_hbm, idx_hbm, out_hbm, idx_v, rows_v, sem):
        wid = lax.axis_index("s") * _NC + lax.axis_index("c")
        base = wid * _B_PER_W
        for c in range(_NCHUNK):
            lo = base + c * _CHUNK
            pltpu.sync_copy(idx_hbm.at[pl.ds(lo, _CHUNK)], idx_v)
            pltpu.async_copy(table_hbm.at[idx_v], rows_v, sem).wait()
            pltpu.sync_copy(rows_v, out_hbm.at[pl.ds(lo, _CHUNK)])

    return gather_k(table, idx)


def kernel(x, W, b, embed):
    wt = W.T
    b2 = b.reshape(1, _DIM)
    ind_blocks, diff = _tc_search(x, wt, b2, embed)
    embed_ind = ind_blocks.reshape(_N)
    quantize = _sc_gather(embed.T, embed_ind)
    return (quantize, diff[0, 0], embed_ind)


# speedup vs baseline: 1.4998x; 1.2446x over previous
import functools

import jax
import jax.numpy as jnp
from jax import lax
from jax.experimental import pallas as pl
from jax.experimental.pallas import tpu as pltpu
from jax.experimental.pallas import tpu_sc as plsc

_N = 16384
_HIDDEN = 768
_DIM = 256
_N_EMBED = 8192

_BN = 256
_NUM_BLOCKS = _N // _BN
_CHUNKS = ((0, 2816), (2816, 5632), (5632, 8192))


def _tc_body(x_ref, wt_ref, b_ref, embed_ref, ind_ref, diff_ref,
             embbf_ref, e2_ref):
    i = pl.program_id(0)
    bf = jnp.bfloat16

    @pl.when(i == 0)
    def _prep():
        emb = embed_ref[...]
        embbf_ref[...] = emb.astype(bf)
        e2_ref[...] = jnp.sum(emb * emb, axis=0, keepdims=True)

    def dotc(lo, sz):
        return jnp.dot(x_ref[:, lo:lo + sz].astype(bf),
                       wt_ref[lo:lo + sz, :].astype(bf),
                       preferred_element_type=jnp.float32)

    xe = dotc(0, 256) + dotc(256, 256) + dotc(512, 256)
    xe = xe + b_ref[...]

    xe_b = xe.astype(bf)
    xe2 = jnp.sum(xe * xe, axis=1, keepdims=True)

    acc_v = acc_i = acc_raw = None
    for lo, hi in _CHUNKS:
        s = jnp.dot(xe_b, embbf_ref[:, lo:hi],
                    preferred_element_type=jnp.float32)
        dist = (xe2 - 2.0 * s) + e2_ref[:, lo:hi]
        m = jnp.min(dist, axis=1, keepdims=True)
        iota = (lax.broadcasted_iota(jnp.int32, dist.shape, 1)
                .astype(jnp.float32) + float(lo))
        idx = jnp.min(jnp.where(dist <= m, iota, float(_N_EMBED)), axis=1,
                      keepdims=True)
        mq = m.astype(bf).astype(jnp.float32)
        if acc_v is None:
            acc_v, acc_i, acc_raw = mq, idx, m
        else:
            keep = acc_v <= m
            acc_i = jnp.where(keep, acc_i, idx)
            acc_raw = jnp.where(keep, acc_raw, m)
            acc_v = jnp.where(keep, acc_v, mq)

    ind_ref[...] = acc_i.astype(jnp.int32).reshape(1, 1, _BN)

    @pl.when(i == 0)
    def _init():
        diff_ref[...] = jnp.zeros_like(diff_ref)

    diff_ref[...] += jnp.sum(acc_raw).reshape(1, 1) * (1.0 / (_N * _DIM))


def _tc_search(x, wt, b2, embed):
    return pl.pallas_call(
        _tc_body,
        grid=(_NUM_BLOCKS,),
        in_specs=[
            pl.BlockSpec((_BN, _HIDDEN), lambda i: (i, 0)),
            pl.BlockSpec((_HIDDEN, _DIM), lambda i: (0, 0)),
            pl.BlockSpec((1, _DIM), lambda i: (0, 0)),
            pl.BlockSpec((_DIM, _N_EMBED), lambda i: (0, 0)),
        ],
        out_specs=[
            pl.BlockSpec((1, 1, _BN), lambda i: (i, 0, 0)),
            pl.BlockSpec((1, 1), lambda i: (0, 0)),
        ],
        out_shape=[
            jax.ShapeDtypeStruct((_NUM_BLOCKS, 1, _BN), jnp.int32),
            jax.ShapeDtypeStruct((1, 1), jnp.float32),
        ],
        scratch_shapes=[
            pltpu.VMEM((_DIM, _N_EMBED), jnp.bfloat16),
            pltpu.VMEM((1, _N_EMBED), jnp.float32),
        ],
        compiler_params=pltpu.CompilerParams(
            dimension_semantics=("arbitrary",),
        ),
    )(x, wt, b2, embed)


try:
    _SC_INFO = plsc.get_sparse_core_info()
    _NC, _NS = _SC_INFO.num_cores, _SC_INFO.num_subcores
except Exception:
    _NC, _NS = 2, 16
_NW = _NC * _NS
_B_PER_W = _N // _NW
_CHUNK = 128
_NCHUNK = _B_PER_W // _CHUNK


def _sc_gather(table, idx):
    mesh = plsc.VectorSubcoreMesh(core_axis_name="c", subcore_axis_name="s")

    @functools.partial(
        pl.kernel,
        mesh=mesh,
        out_type=jax.ShapeDtypeStruct((_N, _DIM), jnp.float32),
        scratch_types=[
            pltpu.VMEM((_CHUNK,), jnp.int32),
            pltpu.VMEM((_CHUNK, _DIM), jnp.float32),
            pltpu.SemaphoreType.DMA,
        ],
    )
    def gather_k(table_hbm, idx_hbm, out_hbm, idx_v, rows_v, sem):
        wid = lax.axis_index("s") * _NC + lax.axis_index("c")
        base = wid * _B_PER_W
        for c in range(_NCHUNK):
            lo = base + c * _CHUNK
            pltpu.sync_copy(idx_hbm.at[pl.ds(lo, _CHUNK)], idx_v)
            pltpu.async_copy(table_hbm.at[idx_v], rows_v, sem).wait()
            pltpu.sync_copy(rows_v, out_hbm.at[pl.ds(lo, _CHUNK)])

    return gather_k(table, idx)


def kernel(x, W, b, embed):
    wt = W.T
    b2 = b.reshape(1, _DIM)
    ind_blocks, diff = _tc_search(x, wt, b2, embed)
    embed_ind = ind_blocks.reshape(_N)
    quantize = _sc_gather(embed.T, embed_ind)
    return (quantize, diff[0, 0], embed_ind)


# speedup vs baseline: 1.5044x; 1.0030x over previous
import functools

import jax
import jax.numpy as jnp
import numpy as np
from jax import lax
from jax.experimental import pallas as pl
from jax.experimental.pallas import tpu as pltpu
from jax.experimental.pallas import tpu_sc as plsc

_N = 16384
_HIDDEN = 768
_DIM = 256
_N_EMBED = 8192

_BN = 256
_NUM_BLOCKS = _N // _BN
_CHUNKS = ((0, 2816), (2816, 5632), (5632, 8192))


def _prep_body(embed_ref, embbf_ref, e2_ref):
    emb = embed_ref[...]
    embbf_ref[...] = emb.astype(jnp.bfloat16)
    e2_ref[...] = jnp.sum(emb * emb, axis=0, keepdims=True)


def _prep(embed):
    return pl.pallas_call(
        _prep_body,
        out_shape=[
            jax.ShapeDtypeStruct((_DIM, _N_EMBED), jnp.bfloat16),
            jax.ShapeDtypeStruct((1, _N_EMBED), jnp.float32),
        ],
    )(embed)


def _tc_body(x_ref, wt_ref, b_ref, embbf_ref, e2_ref, iota_ref,
             ind_ref, diff_ref):
    i = pl.program_id(0)
    bf = jnp.bfloat16

    def dotc(lo, sz):
        return jnp.dot(x_ref[:, lo:lo + sz].astype(bf),
                       wt_ref[lo:lo + sz, :].astype(bf),
                       preferred_element_type=jnp.float32)

    xe = dotc(0, 256) + dotc(256, 256) + dotc(512, 256)
    xe = xe + b_ref[...]

    xe_b = xe.astype(bf)
    xe2 = jnp.sum(xe * xe, axis=1, keepdims=True)

    acc_v = acc_i = acc_raw = None
    for lo, hi in _CHUNKS:
        s = jnp.dot(xe_b, embbf_ref[:, lo:hi],
                    preferred_element_type=jnp.float32)
        dist = (xe2 - 2.0 * s) + e2_ref[:, lo:hi]
        m = jnp.min(dist, axis=1, keepdims=True)
        idx = jnp.min(jnp.where(dist <= m, iota_ref[:, lo:hi],
                                float(_N_EMBED)), axis=1, keepdims=True)
        mq = m.astype(bf).astype(jnp.float32)
        if acc_v is None:
            acc_v, acc_i, acc_raw = mq, idx, m
        else:
            keep = acc_v <= m
            acc_i = jnp.where(keep, acc_i, idx)
            acc_raw = jnp.where(keep, acc_raw, m)
            acc_v = jnp.where(keep, acc_v, mq)

    ind_ref[...] = acc_i.astype(jnp.int32).reshape(1, 1, _BN)

    @pl.when(i == 0)
    def _init():
        diff_ref[...] = jnp.zeros_like(diff_ref)

    diff_ref[...] += jnp.sum(acc_raw).reshape(1, 1) * (1.0 / (_N * _DIM))


def _tc_search(x, wt, b2, embbf, e2, iota):
    return pl.pallas_call(
        _tc_body,
        grid=(_NUM_BLOCKS,),
        in_specs=[
            pl.BlockSpec((_BN, _HIDDEN), lambda i: (i, 0)),
            pl.BlockSpec((_HIDDEN, _DIM), lambda i: (0, 0)),
            pl.BlockSpec((1, _DIM), lambda i: (0, 0)),
            pl.BlockSpec((_DIM, _N_EMBED), lambda i: (0, 0)),
            pl.BlockSpec((1, _N_EMBED), lambda i: (0, 0)),
            pl.BlockSpec((1, _N_EMBED), lambda i: (0, 0)),
        ],
        out_specs=[
            pl.BlockSpec((1, 1, _BN), lambda i: (i, 0, 0)),
            pl.BlockSpec((1, 1), lambda i: (0, 0)),
        ],
        out_shape=[
            jax.ShapeDtypeStruct((_NUM_BLOCKS, 1, _BN), jnp.int32),
            jax.ShapeDtypeStruct((1, 1), jnp.float32),
        ],
        compiler_params=pltpu.CompilerParams(
            dimension_semantics=("arbitrary",),
        ),
    )(x, wt, b2, embbf, e2, iota)


try:
    _SC_INFO = plsc.get_sparse_core_info()
    _NC, _NS = _SC_INFO.num_cores, _SC_INFO.num_subcores
except Exception:
    _NC, _NS = 2, 16
_NW = _NC * _NS
_B_PER_W = _N // _NW
_CHUNK = 128
_NCHUNK = _B_PER_W // _CHUNK


def _sc_gather(table, idx):
    mesh = plsc.VectorSubcoreMesh(core_axis_name="c", subcore_axis_name="s")

    @functools.partial(
        pl.kernel,
        mesh=mesh,
        out_type=jax.ShapeDtypeStruct((_N, _DIM), jnp.float32),
        scratch_types=[
            pltpu.VMEM((_CHUNK,), jnp.int32),
            pltpu.VMEM((_CHUNK, _DIM), jnp.float32),
            pltpu.SemaphoreType.DMA,
        ],
    )
    def gather_k(table_hbm, idx_hbm, out_hbm, idx_v, rows_v, sem):
        wid = lax.axis_index("s") * _NC + lax.axis_index("c")
        base = wid * _B_PER_W
        for c in range(_NCHUNK):
            lo = base + c * _CHUNK
            pltpu.sync_copy(idx_hbm.at[pl.ds(lo, _CHUNK)], idx_v)
            pltpu.async_copy(table_hbm.at[idx_v], rows_v, sem).wait()
            pltpu.sync_copy(rows_v, out_hbm.at[pl.ds(lo, _CHUNK)])

    return gather_k(table, idx)


_IOTA = np.arange(_N_EMBED, dtype=np.float32).reshape(1, _N_EMBED)


def kernel(x, W, b, embed):
    wt = W.T
    b2 = b.reshape(1, _DIM)
    embbf, e2 = _prep(embed)
    iota = jnp.asarray(_IOTA)
    ind_blocks, diff = _tc_search(x, wt, b2, embbf, e2, iota)
    embed_ind = ind_blocks.reshape(_N)
    quantize = _sc_gather(embed.T, embed_ind)
    return (quantize, diff[0, 0], embed_ind)
